# baseline (speedup 1.0000x reference)
import functools

import jax
import jax.numpy as jnp
from jax import lax
from jax.experimental import pallas as pl
from jax.experimental.pallas import tpu as pltpu
from jax.experimental.pallas import tpu_sc as plsc

N = 10000
E = 320000
D = 128
H = 32

BR = 200
NC = 2
NS = 16
NW = NC * NS
EPW = E // NW
CH = 400
L = 16


def _enc_body(x_ref, w_ref, out_ref):
    out_ref[...] = jnp.dot(x_ref[...], w_ref[...],
                           preferred_element_type=jnp.float32)


def _node_body(a_ref, b_ref, w1t_ref, w1b_ref, b1_ref, ne_ref, p_ref, q_ref):
    ne = jnp.maximum(
        jnp.dot(a_ref[...], b_ref[...], preferred_element_type=jnp.float32),
        0.0)
    ne_ref[...] = ne
    p_ref[...] = jnp.dot(ne, w1t_ref[...],
                         preferred_element_type=jnp.float32) + b1_ref[...]
    q_ref[...] = jnp.dot(ne, w1b_ref[...],
                         preferred_element_type=jnp.float32)


def _gumbel_body(lg_ref, eps_ref, u_ref, b2_ref, adj_ref):
    logit = lg_ref[...] + b2_ref[...]
    eps = 0.9999 - 0.9998 * eps_ref[...]
    gate = jax.nn.sigmoid(jnp.log(eps) - jnp.log(1.0 - eps) + logit)
    att = jnp.clip(gate, 0.01, 0.99)
    lo = jnp.log(att) - jnp.log1p(-att)
    u = jnp.clip(u_ref[...], 1e-6, 1.0 - 1e-6)
    w = jax.nn.sigmoid((lo + jnp.log(u) - jnp.log(1.0 - u)) / 0.9)
    adj_ref[...] = w * (w > 0.2).astype(jnp.float32)


def _sc_edge_body(p_hbm, q_hbm, src_hbm, dst_hbm, w2_hbm, lg_out,
                  si_a, di_a, si_b, di_b, pr_a, qr_a, pr_b, qr_b,
                  hs_v, lg_v, w2_v,
                  sem_pa, sem_qa, sem_pb, sem_qb):
    wid = lax.axis_index("s") * NC + lax.axis_index("c")
    base = wid * EPW
    nch = EPW // CH
    pltpu.sync_copy(w2_hbm, w2_v)
    w2s = [w2_v[j, :] for j in range(H)]

    def issue(c, si, di, pr, qr, sp, sq):
        off = base + c * CH
        pltpu.sync_copy(src_hbm.at[pl.ds(off, CH)], si)
        pltpu.sync_copy(dst_hbm.at[pl.ds(off, CH)], di)
        pltpu.async_copy(p_hbm.at[si], pr, sp)
        pltpu.async_copy(q_hbm.at[di], qr, sq)

    def wait_for(si, di, pr, qr, sp, sq):
        pltpu.make_async_copy(p_hbm.at[si], pr, sp).wait()
        pltpu.make_async_copy(q_hbm.at[di], qr, sq).wait()

    def compute(c, pr_v, qr_v):
        @plsc.parallel_loop(0, CH, 1, unroll=4)
        def pre_body(e):
            h0 = jnp.maximum(pr_v[e, pl.ds(0, L)] + qr_v[e, pl.ds(0, L)], 0.0)
            h1 = jnp.maximum(pr_v[e, pl.ds(L, L)] + qr_v[e, pl.ds(L, L)], 0.0)
            hs_v[e, pl.ds(0, L)] = h0
            hs_v[e, pl.ds(L, L)] = h1

        @plsc.parallel_loop(0, CH // L, 1, unroll=2)
        def group_body(g):
            rows = g * L + lax.iota(jnp.int32, L)
            acc = jnp.zeros((L,), jnp.float32)
            for j in range(H):
                col = jnp.full((L,), j, jnp.int32)
                h = plsc.load_gather(hs_v, [rows, col])
                acc = acc + h * w2s[j]
            lg_v[pl.ds(g * L, L)] = acc
        pltpu.sync_copy(lg_v, lg_out.at[pl.ds(base + c * CH, CH)])

    buf_a = (si_a, di_a, pr_a, qr_a, sem_pa, sem_qa)
    buf_b = (si_b, di_b, pr_b, qr_b, sem_pb, sem_qb)

    issue(0, *buf_a)

    def pair(i, carry):
        cc = 2 * i
        wait_for(*buf_a)
        issue(cc + 1, *buf_b)
        compute(cc, pr_a, qr_a)
        wait_for(*buf_b)
        issue(cc + 2, *buf_a)
        compute(cc + 1, pr_b, qr_b)
        return carry

    lax.fori_loop(0, (nch - 1) // 2, pair, 0)
    wait_for(*buf_a)
    compute(nch - 1, pr_a, qr_a)


def _make_edge_kernel():
    mesh = plsc.VectorSubcoreMesh(core_axis_name="c", subcore_axis_name="s")
    return pl.kernel(
        _sc_edge_body,
        mesh=mesh,
        out_type=[jax.ShapeDtypeStruct((E,), jnp.float32)],
        scratch_types=[pltpu.VMEM((CH,), jnp.int32),
                       pltpu.VMEM((CH,), jnp.int32),
                       pltpu.VMEM((CH,), jnp.int32),
                       pltpu.VMEM((CH,), jnp.int32),
                       pltpu.VMEM((CH, H), jnp.float32),
                       pltpu.VMEM((CH, H), jnp.float32),
                       pltpu.VMEM((CH, H), jnp.float32),
                       pltpu.VMEM((CH, H), jnp.float32),
                       pltpu.VMEM((CH, H + 1), jnp.float32),
                       pltpu.VMEM((CH,), jnp.float32),
                       pltpu.VMEM((H, L), jnp.float32),
                       pltpu.SemaphoreType.DMA,
                       pltpu.SemaphoreType.DMA,
                       pltpu.SemaphoreType.DMA,
                       pltpu.SemaphoreType.DMA],
        compiler_params=pltpu.CompilerParams(use_tc_tiling_on_sc=False,
                                             needs_layout_passes=False),
    )


def _impl(x, edge_index, norm_adjacent_matrix, W_enc, W1, b1, W2, b2,
          eps_noise, u_noise):
    b_mat = pl.pallas_call(
        _enc_body,
        out_shape=jax.ShapeDtypeStruct((N, H), jnp.float32),
    )(x, W_enc)

    w1t = W1[:H]
    w1b = W1[H:]
    b1r = b1.reshape(1, H)
    grid = N // BR
    node_emb, p_tab, q_tab = pl.pallas_call(
        _node_body,
        grid=(grid,),
        in_specs=[
            pl.BlockSpec((BR, N), lambda i: (i, 0)),
            pl.BlockSpec((N, H), lambda i: (0, 0)),
            pl.BlockSpec((H, H), lambda i: (0, 0)),
            pl.BlockSpec((H, H), lambda i: (0, 0)),
            pl.BlockSpec((1, H), lambda i: (0, 0)),
        ],
        out_specs=[
            pl.BlockSpec((BR, H), lambda i: (i, 0)),
            pl.BlockSpec((BR, H), lambda i: (i, 0)),
            pl.BlockSpec((BR, H), lambda i: (i, 0)),
        ],
        out_shape=[
            jax.ShapeDtypeStruct((N, H), jnp.float32),
            jax.ShapeDtypeStruct((N, H), jnp.float32),
            jax.ShapeDtypeStruct((N, H), jnp.float32),
        ],
        compiler_params=pltpu.CompilerParams(
            vmem_limit_bytes=120 * 1024 * 1024),
    )(norm_adjacent_matrix, b_mat, w1t, w1b, b1r)

    src = edge_index[0]
    dst = edge_index[1]
    w2b = jnp.tile(W2.reshape(H, 1), (1, L))
    (logits,) = _make_edge_kernel()(p_tab, q_tab, src, dst, w2b)

    lg2 = logits.reshape(E // 128, 128)
    eps2 = eps_noise.reshape(E // 128, 128)
    u2 = u_noise.reshape(E // 128, 128)
    b2r = b2.reshape(1, 1)
    adj2 = pl.pallas_call(
        _gumbel_body,
        out_shape=jax.ShapeDtypeStruct((E // 128, 128), jnp.float32),
    )(lg2, eps2, u2, b2r)
    adj = adj2.reshape(E)

    return (node_emb, adj)


kernel = _impl


# speedup vs baseline: 1.1379x; 1.1379x over previous
import functools

import jax
import jax.numpy as jnp
from jax import lax
from jax.experimental import pallas as pl
from jax.experimental.pallas import tpu as pltpu
from jax.experimental.pallas import tpu_sc as plsc

N = 10000
E = 320000
D = 128
H = 32

BR = 400
NC = 2
NS = 16
NW = NC * NS
EPW = E // NW
CH = 400
L = 16


def _enc_body(x_ref, w_ref, out_ref):
    out_ref[...] = jnp.dot(x_ref[...], w_ref[...],
                           preferred_element_type=jnp.float32)


def _node_body(a_ref, b_ref, w1t_ref, w1b_ref, b1_ref, ne_ref, p_ref, q_ref):
    ne = jnp.maximum(
        jnp.dot(a_ref[...], b_ref[...], preferred_element_type=jnp.float32),
        0.0)
    ne_ref[...] = ne
    p_ref[...] = (jnp.dot(ne, w1t_ref[...], preferred_element_type=jnp.float32)
                  + b1_ref[...]).astype(jnp.bfloat16)
    q_ref[...] = jnp.dot(ne, w1b_ref[...],
                         preferred_element_type=jnp.float32).astype(jnp.bfloat16)


def _gumbel_body(lg_ref, eps_ref, u_ref, b2_ref, adj_ref):
    logit = lg_ref[...] + b2_ref[...]
    eps = 0.9999 - 0.9998 * eps_ref[...]
    gate = jax.nn.sigmoid(jnp.log(eps) - jnp.log(1.0 - eps) + logit)
    att = jnp.clip(gate, 0.01, 0.99)
    lo = jnp.log(att) - jnp.log1p(-att)
    u = jnp.clip(u_ref[...], 1e-6, 1.0 - 1e-6)
    w = jax.nn.sigmoid((lo + jnp.log(u) - jnp.log(1.0 - u)) / 0.9)
    adj_ref[...] = w * (w > 0.2).astype(jnp.float32)


def _sc_edge_body(p_hbm, q_hbm, src_hbm, dst_hbm, w2_hbm, lg_out,
                  si_a, di_a, si_b, di_b, pr_a, qr_a, pr_b, qr_b,
                  hs_v, lg_v, w2_v,
                  sem_pa, sem_qa, sem_pb, sem_qb):
    wid = lax.axis_index("s") * NC + lax.axis_index("c")
    base = wid * EPW
    nch = EPW // CH
    pltpu.sync_copy(w2_hbm, w2_v)
    w2s = [w2_v[j, :] for j in range(H)]

    def issue(c, si, di, pr, qr, sp, sq):
        off = base + c * CH
        pltpu.sync_copy(src_hbm.at[pl.ds(off, CH)], si)
        pltpu.sync_copy(dst_hbm.at[pl.ds(off, CH)], di)
        pltpu.async_copy(p_hbm.at[si], pr, sp)
        pltpu.async_copy(q_hbm.at[di], qr, sq)

    def wait_for(si, di, pr, qr, sp, sq):
        pltpu.make_async_copy(p_hbm.at[si], pr, sp).wait()
        pltpu.make_async_copy(q_hbm.at[di], qr, sq).wait()

    def compute(c, pr_v, qr_v):
        @plsc.parallel_loop(0, CH, 1, unroll=4)
        def pre_body(e):
            s = jnp.maximum(pr_v[e, pl.ds(0, H)] + qr_v[e, pl.ds(0, H)],
                            jnp.bfloat16(0))
            hs_v[e, pl.ds(0, L)] = plsc.bitcast(s, jnp.int32)

        @plsc.parallel_loop(0, CH // L, 1, unroll=2)
        def group_body(g):
            rows = g * L + lax.iota(jnp.int32, L)
            acc = jnp.zeros((L,), jnp.float32)
            for j in range(H // 2):
                col = jnp.full((L,), j, jnp.int32)
                pair = plsc.load_gather(hs_v, [rows, col])
                lo, hi = plsc.unpack(plsc.bitcast(pair, jnp.bfloat16),
                                     format=plsc.PackFormat.INTERLEAVED)
                acc = acc + lo * w2s[2 * j] + hi * w2s[2 * j + 1]
            lg_v[pl.ds(g * L, L)] = acc
        pltpu.sync_copy(lg_v, lg_out.at[pl.ds(base + c * CH, CH)])

    buf_a = (si_a, di_a, pr_a, qr_a, sem_pa, sem_qa)
    buf_b = (si_b, di_b, pr_b, qr_b, sem_pb, sem_qb)

    issue(0, *buf_a)

    def pair(i, carry):
        cc = 2 * i
        wait_for(*buf_a)
        issue(cc + 1, *buf_b)
        compute(cc, pr_a, qr_a)
        wait_for(*buf_b)
        issue(cc + 2, *buf_a)
        compute(cc + 1, pr_b, qr_b)
        return carry

    lax.fori_loop(0, (nch - 1) // 2, pair, 0)
    wait_for(*buf_a)
    compute(nch - 1, pr_a, qr_a)


def _make_edge_kernel():
    mesh = plsc.VectorSubcoreMesh(core_axis_name="c", subcore_axis_name="s")
    return pl.kernel(
        _sc_edge_body,
        mesh=mesh,
        out_type=[jax.ShapeDtypeStruct((E,), jnp.float32)],
        scratch_types=[pltpu.VMEM((CH,), jnp.int32),
                       pltpu.VMEM((CH,), jnp.int32),
                       pltpu.VMEM((CH,), jnp.int32),
                       pltpu.VMEM((CH,), jnp.int32),
                       pltpu.VMEM((CH, H), jnp.bfloat16),
                       pltpu.VMEM((CH, H), jnp.bfloat16),
                       pltpu.VMEM((CH, H), jnp.bfloat16),
                       pltpu.VMEM((CH, H), jnp.bfloat16),
                       pltpu.VMEM((CH, H // 2 + 1), jnp.int32),
                       pltpu.VMEM((CH,), jnp.float32),
                       pltpu.VMEM((H, L), jnp.float32),
                       pltpu.SemaphoreType.DMA,
                       pltpu.SemaphoreType.DMA,
                       pltpu.SemaphoreType.DMA,
                       pltpu.SemaphoreType.DMA],
        compiler_params=pltpu.CompilerParams(use_tc_tiling_on_sc=False,
                                             needs_layout_passes=False),
    )


def _impl(x, edge_index, norm_adjacent_matrix, W_enc, W1, b1, W2, b2,
          eps_noise, u_noise):
    b_mat = pl.pallas_call(
        _enc_body,
        out_shape=jax.ShapeDtypeStruct((N, H), jnp.float32),
    )(x, W_enc)

    w1t = W1[:H]
    w1b = W1[H:]
    b1r = b1.reshape(1, H)
    grid = N // BR
    node_emb, p_tab, q_tab = pl.pallas_call(
        _node_body,
        grid=(grid,),
        in_specs=[
            pl.BlockSpec((BR, N), lambda i: (i, 0)),
            pl.BlockSpec((N, H), lambda i: (0, 0)),
            pl.BlockSpec((H, H), lambda i: (0, 0)),
            pl.BlockSpec((H, H), lambda i: (0, 0)),
            pl.BlockSpec((1, H), lambda i: (0, 0)),
        ],
        out_specs=[
            pl.BlockSpec((BR, H), lambda i: (i, 0)),
            pl.BlockSpec((BR, H), lambda i: (i, 0)),
            pl.BlockSpec((BR, H), lambda i: (i, 0)),
        ],
        out_shape=[
            jax.ShapeDtypeStruct((N, H), jnp.float32),
            jax.ShapeDtypeStruct((N, H), jnp.bfloat16),
            jax.ShapeDtypeStruct((N, H), jnp.bfloat16),
        ],
        compiler_params=pltpu.CompilerParams(
            vmem_limit_bytes=120 * 1024 * 1024),
    )(norm_adjacent_matrix, b_mat, w1t, w1b, b1r)

    src = edge_index[0]
    dst = edge_index[1]
    w2b = jnp.tile(W2.reshape(H, 1), (1, L))
    (logits,) = _make_edge_kernel()(p_tab, q_tab, src, dst, w2b)

    lg2 = logits.reshape(E // 128, 128)
    eps2 = eps_noise.reshape(E // 128, 128)
    u2 = u_noise.reshape(E // 128, 128)
    b2r = b2.reshape(1, 1)
    adj2 = pl.pallas_call(
        _gumbel_body,
        out_shape=jax.ShapeDtypeStruct((E // 128, 128), jnp.float32),
    )(lg2, eps2, u2, b2r)
    adj = adj2.reshape(E)

    return (node_emb, adj)


kernel = _impl


# speedup vs baseline: 1.2095x; 1.0629x over previous
import functools

import jax
import jax.numpy as jnp
from jax import lax
from jax.experimental import pallas as pl
from jax.experimental.pallas import tpu as pltpu
from jax.experimental.pallas import tpu_sc as plsc

N = 10000
E = 320000
D = 128
H = 32

BR = 400
NC = 2
NS = 16
NW = NC * NS
EPW = E // NW
CH = 1000
L = 16
CHP = CH + 8
NG = (CH + L - 1) // L


def _enc_body(x_ref, w_ref, out_ref):
    out_ref[...] = jnp.dot(x_ref[...], w_ref[...],
                           preferred_element_type=jnp.float32)


def _node_body(a_ref, b_ref, w1t_ref, w1b_ref, b1_ref, ne_ref, p_ref, q_ref):
    ne = jnp.maximum(
        jnp.dot(a_ref[...], b_ref[...], preferred_element_type=jnp.float32),
        0.0)
    ne_ref[...] = ne
    p_ref[...] = (jnp.dot(ne, w1t_ref[...], preferred_element_type=jnp.float32)
                  + b1_ref[...]).astype(jnp.bfloat16)
    q_ref[...] = jnp.dot(ne, w1b_ref[...],
                         preferred_element_type=jnp.float32).astype(jnp.bfloat16)


def _gumbel_body(lg_ref, eps_ref, u_ref, b2_ref, adj_ref):
    logit = lg_ref[...] + b2_ref[...]
    eps = 0.9999 - 0.9998 * eps_ref[...]
    gate = jax.nn.sigmoid(jnp.log(eps) - jnp.log(1.0 - eps) + logit)
    att = jnp.clip(gate, 0.01, 0.99)
    lo = jnp.log(att) - jnp.log1p(-att)
    u = jnp.clip(u_ref[...], 1e-6, 1.0 - 1e-6)
    w = jax.nn.sigmoid((lo + jnp.log(u) - jnp.log(1.0 - u)) / 0.9)
    adj_ref[...] = w * (w > 0.2).astype(jnp.float32)


def _sc_edge_body(p_hbm, q_hbm, src_hbm, dst_hbm, w2_hbm, lg_out,
                  si_a, di_a, si_b, di_b, pr_a, qr_a, pr_b, qr_b,
                  hs_v, lg_v, w2_v,
                  sem_pa, sem_qa, sem_pb, sem_qb):
    wid = lax.axis_index("s") * NC + lax.axis_index("c")
    base = wid * EPW
    nch = EPW // CH
    pltpu.sync_copy(w2_hbm, w2_v)
    w2s = [w2_v[j, :] for j in range(H)]

    for r in range(CH, CHP):
        hs_v[r, pl.ds(0, L)] = jnp.zeros((L,), jnp.int32)

    def issue(c, si, di, pr, qr, sp, sq):
        off = base + c * CH
        pltpu.sync_copy(src_hbm.at[pl.ds(off, CH)], si)
        pltpu.sync_copy(dst_hbm.at[pl.ds(off, CH)], di)
        pltpu.async_copy(p_hbm.at[si], pr, sp)
        pltpu.async_copy(q_hbm.at[di], qr, sq)

    def wait_for(si, di, pr, qr, sp, sq):
        pltpu.make_async_copy(p_hbm.at[si], pr, sp).wait()
        pltpu.make_async_copy(q_hbm.at[di], qr, sq).wait()

    def compute(c, pr_v, qr_v):
        @plsc.parallel_loop(0, CH, 1, unroll=4)
        def pre_body(e):
            s = jnp.maximum(pr_v[e, pl.ds(0, H)] + qr_v[e, pl.ds(0, H)],
                            jnp.bfloat16(0))
            hs_v[e, pl.ds(0, L)] = plsc.bitcast(s, jnp.int32)

        @plsc.parallel_loop(0, NG, 1, unroll=2)
        def group_body(g):
            rows = g * L + lax.iota(jnp.int32, L)
            acc = jnp.zeros((L,), jnp.float32)
            for j in range(H // 2):
                col = jnp.full((L,), j, jnp.int32)
                pair = plsc.load_gather(hs_v, [rows, col])
                lo, hi = plsc.unpack(plsc.bitcast(pair, jnp.bfloat16),
                                     format=plsc.PackFormat.INTERLEAVED)
                acc = acc + lo * w2s[2 * j] + hi * w2s[2 * j + 1]
            lg_v[pl.ds(g * L, L)] = acc
        pltpu.sync_copy(lg_v.at[pl.ds(0, CH)],
                        lg_out.at[pl.ds(base + c * CH, CH)])

    buf_a = (si_a, di_a, pr_a, qr_a, sem_pa, sem_qa)
    buf_b = (si_b, di_b, pr_b, qr_b, sem_pb, sem_qb)

    issue(0, *buf_a)

    def pair(i, carry):
        cc = 2 * i
        wait_for(*buf_a)
        issue(cc + 1, *buf_b)
        compute(cc, pr_a, qr_a)
        wait_for(*buf_b)

        @pl.when(cc + 2 < nch)
        def _():
            issue(cc + 2, *buf_a)

        compute(cc + 1, pr_b, qr_b)
        return carry

    lax.fori_loop(0, nch // 2, pair, 0)


def _make_edge_kernel():
    mesh = plsc.VectorSubcoreMesh(core_axis_name="c", subcore_axis_name="s")
    return pl.kernel(
        _sc_edge_body,
        mesh=mesh,
        out_type=[jax.ShapeDtypeStruct((E,), jnp.float32)],
        scratch_types=[pltpu.VMEM((CH,), jnp.int32),
                       pltpu.VMEM((CH,), jnp.int32),
                       pltpu.VMEM((CH,), jnp.int32),
                       pltpu.VMEM((CH,), jnp.int32),
                       pltpu.VMEM((CH, H), jnp.bfloat16),
                       pltpu.VMEM((CH, H), jnp.bfloat16),
                       pltpu.VMEM((CH, H), jnp.bfloat16),
                       pltpu.VMEM((CH, H), jnp.bfloat16),
                       pltpu.VMEM((CHP, H // 2 + 1), jnp.int32),
                       pltpu.VMEM((CHP,), jnp.float32),
                       pltpu.VMEM((H, L), jnp.float32),
                       pltpu.SemaphoreType.DMA,
                       pltpu.SemaphoreType.DMA,
                       pltpu.SemaphoreType.DMA,
                       pltpu.SemaphoreType.DMA],
        compiler_params=pltpu.CompilerParams(use_tc_tiling_on_sc=False,
                                             needs_layout_passes=False),
    )


def _impl(x, edge_index, norm_adjacent_matrix, W_enc, W1, b1, W2, b2,
          eps_noise, u_noise):
    b_mat = pl.pallas_call(
        _enc_body,
        out_shape=jax.ShapeDtypeStruct((N, H), jnp.float32),
    )(x, W_enc)

    w1t = W1[:H]
    w1b = W1[H:]
    b1r = b1.reshape(1, H)
    grid = N // BR
    node_emb, p_tab, q_tab = pl.pallas_call(
        _node_body,
        grid=(grid,),
        in_specs=[
            pl.BlockSpec((BR, N), lambda i: (i, 0)),
            pl.BlockSpec((N, H), lambda i: (0, 0)),
            pl.BlockSpec((H, H), lambda i: (0, 0)),
            pl.BlockSpec((H, H), lambda i: (0, 0)),
            pl.BlockSpec((1, H), lambda i: (0, 0)),
        ],
        out_specs=[
            pl.BlockSpec((BR, H), lambda i: (i, 0)),
            pl.BlockSpec((BR, H), lambda i: (i, 0)),
            pl.BlockSpec((BR, H), lambda i: (i, 0)),
        ],
        out_shape=[
            jax.ShapeDtypeStruct((N, H), jnp.float32),
            jax.ShapeDtypeStruct((N, H), jnp.bfloat16),
            jax.ShapeDtypeStruct((N, H), jnp.bfloat16),
        ],
        compiler_params=pltpu.CompilerParams(
            vmem_limit_bytes=120 * 1024 * 1024),
    )(norm_adjacent_matrix, b_mat, w1t, w1b, b1r)

    src = edge_index[0]
    dst = edge_index[1]
    w2b = jnp.tile(W2.reshape(H, 1), (1, L))
    (logits,) = _make_edge_kernel()(p_tab, q_tab, src, dst, w2b)

    lg2 = logits.reshape(E // 128, 128)
    eps2 = eps_noise.reshape(E // 128, 128)
    u2 = u_noise.reshape(E // 128, 128)
    b2r = b2.reshape(1, 1)
    adj2 = pl.pallas_call(
        _gumbel_body,
        out_shape=jax.ShapeDtypeStruct((E // 128, 128), jnp.float32),
    )(lg2, eps2, u2, b2r)
    adj = adj2.reshape(E)

    return (node_emb, adj)


kernel = _impl
